# baseline (speedup 1.0000x reference)
import functools

import jax
import jax.numpy as jnp
from jax import lax
from jax.experimental import pallas as pl
from jax.experimental.pallas import tpu as pltpu
from jax.experimental.pallas import tpu_sc as plsc

_LANES = 16


def _build(B, H, W, C, P):
    NC, NS = 2, 16
    n_pts = B * P
    sc_pts = n_pts // NC
    ppw = sc_pts // NS
    nbins = (B // NC) * H
    band = nbins // NS
    slab_w = (C // 8) * (W // 128) * 1024
    cg = C // 8
    assert H % 128 == 0 and W % 128 == 0 and C % 8 == 0 and P % 128 == 0

    mesh = plsc.VectorSubcoreMesh(
        core_axis_name="c", subcore_axis_name="s", num_cores=NC, num_subcores=NS)

    @functools.partial(
        pl.kernel,
        out_type=(jax.ShapeDtypeStruct((n_pts + 16, C), jnp.float32),
                  jax.ShapeDtypeStruct((n_pts + 128, 8), jnp.float32)),
        mesh=mesh,
        compiler_params=pltpu.CompilerParams(
            needs_layout_passes=False, use_tc_tiling_on_sc=False),
        scratch_types=[
            pltpu.VMEM((nbins + 16,), jnp.int32),
            pltpu.VMEM((nbins,), jnp.int32),
            pltpu.SMEM((nbins + 16,), jnp.int32),
            pltpu.VMEM_SHARED((NS, nbins), jnp.int32),
            pltpu.SemaphoreType.DMA,
            pltpu.SemaphoreType.DMA,
        ],
    )
    def point_sample(feat_hbm, grid_hbm, out_hbm, rec_hbm,
                     binstart, cursor, binsmem, shared, sem, sem2):
        cid = lax.axis_index("c")
        sid = lax.axis_index("s")
        b_local = sid // (NS // (B // NC))
        pt0_sc = cid * sc_pts
        chunk0 = pt0_sc + sid * ppw
        lane = lax.iota(jnp.int32, 16)
        ones = jnp.ones((16,), jnp.int32)
        fW = jnp.float32(W)
        fH = jnp.float32(H)

        def floor_i32(f):
            t = f.astype(jnp.int32)
            return t - jnp.where(t.astype(jnp.float32) > f, 1, 0)

        def phase_a(gchunk, hist, histall, recs, pos):
            for v in range(nbins // 16):
                hist[pl.ds(v * 16, 16)] = jnp.zeros((16,), jnp.int32)
            pltpu.sync_copy(grid_hbm.at[pl.ds(chunk0 // 128, ppw // 128)], gchunk)

            def keys_of(v):
                row = v // 8
                o = (v % 8) * 16
                gx = gchunk[row, pl.ds(o, 16)]
                gy = gchunk[row, pl.ds(128 + o, 16)]
                iy0 = floor_i32(gy * fH - 0.5)
                k = b_local * H + jnp.clip(iy0, 0, H - 1)
                return gx, gy, k

            def pass1(v, carry):
                _, _, k = keys_of(v)
                plsc.addupdate_scatter(hist, [k], ones)
                return carry

            lax.fori_loop(0, ppw // 16, pass1, 0)
            pltpu.sync_copy(hist, shared.at[sid])
            plsc.subcore_barrier()
            pltpu.sync_copy(shared, histall)

            def prefix(v, run):
                sl = pl.ds(v * 16, 16)
                tot = jnp.zeros((16,), jnp.int32)
                pre = jnp.zeros((16,), jnp.int32)
                for t in range(NS):
                    hv = histall[t, sl]
                    tot = tot + hv
                    pre = pre + jnp.where(jnp.full((16,), t, jnp.int32)
                                          < jnp.full((16,), sid, jnp.int32),
                                          hv, jnp.zeros((16,), jnp.int32))
                excl = plsc.cumsum(tot) - tot
                bsv = excl + run
                binstart[sl] = bsv
                cursor[sl] = bsv + pre
                for i in range(16):
                    binsmem[v * 16 + i] = bsv[i]
                return run + jnp.sum(tot, axis=0)

            lax.fori_loop(0, nbins // 16, prefix, jnp.int32(0))
            binsmem[nbins] = jnp.int32(sc_pts)

            def pass2(v, carry):
                gx, gy, k = keys_of(v)
                rank, _ = plsc.scan_count(k)
                basek = plsc.load_gather(cursor, [k])
                p = basek + rank - 1
                plsc.addupdate_scatter(cursor, [k], ones)
                pos[v // 8, pl.ds((v % 8) * 16, 16)] = (
                    jnp.clip(p, 0, sc_pts - 1) + pt0_sc)
                pid = (chunk0 + v * 16 + lane).astype(jnp.float32)
                pt = v * 16 + lane
                plsc.store_scatter(recs, [pt, jnp.zeros((16,), jnp.int32)], gx)
                plsc.store_scatter(recs, [pt, ones], gy)
                plsc.store_scatter(recs, [pt, ones + ones], pid)
                return carry

            lax.fori_loop(0, ppw // 16, pass2, 0)
            descs = [
                pltpu.async_copy(recs.at[pl.ds(ch * 128, 128)],
                                 rec_hbm.at[pos.at[ch]], sem)
                for ch in range(ppw // 128)
            ]
            for d in descs:
                d.wait()

        pl.run_scoped(
            phase_a,
            pltpu.VMEM((ppw // 128, 256), jnp.float32),
            pltpu.VMEM((nbins,), jnp.int32),
            pltpu.VMEM((NS, nbins), jnp.int32),
            pltpu.VMEM((ppw, 8), jnp.float32),
            pltpu.VMEM((ppw // 128, 128), jnp.int32),
        )
        plsc.subcore_barrier()

        k0 = sid * band
        y0 = (sid % (NS // (B // NC))) * band
        bb = cid * (B // NC) + b_local
        slab0 = (bb * H) * slab_w

        def phase_b(slab, recst, stag, sbuf, sidx):
            pltpu.sync_copy(
                feat_hbm.at[pl.ds(slab0 + y0 * slab_w, slab_w)],
                slab.at[pl.ds((y0 % 2) * slab_w, slab_w)])
            pltpu.async_copy(
                feat_hbm.at[pl.ds(slab0 + (y0 + 1) * slab_w, slab_w)],
                slab.at[pl.ds(((y0 + 1) % 2) * slab_w, slab_w)], sem2)

            def do_bin(kk, carry):
                y = y0 + kk

                @pl.when(y + 1 <= H - 1)
                def _():
                    pltpu.make_async_copy(
                        feat_hbm.at[pl.ds(0, slab_w)],
                        slab.at[pl.ds(((y + 1) % 2) * slab_w, slab_w)],
                        sem2).wait()

                cur = binsmem[k0 + kk]
                nxt = binsmem[k0 + kk + 1]
                n = jnp.clip(nxt - cur, 0, sc_pts)
                gstart = pt0_sc + cur
                lo_off = jnp.full((16,), (y % 2) * slab_w, jnp.int32)
                hi_off = jnp.full((16,), ((y + 1) % 2) * slab_w, jnp.int32)

                def do_chunk(ch, c2):
                    pltpu.sync_copy(rec_hbm.at[pl.ds(gstart + ch * 64, 64)],
                                    recst)
                    m = jnp.minimum(jnp.int32(64), n - ch * 64)
                    for vv in range(4):
                        sidx[0, pl.ds(vv * 16, 16)] = (
                            jnp.full((16,), n_pts, jnp.int32) + lane)

                    def do_vec(v, c3):
                        pt = v * 16 + lane
                        zz = jnp.zeros((16,), jnp.int32)
                        gx = plsc.load_gather(recst, [pt, zz])
                        gy = plsc.load_gather(recst, [pt, zz + 1])
                        pidf = plsc.load_gather(recst, [pt, zz + 2])
                        fx = gx * fW - 0.5
                        fy = gy * fH - 0.5
                        ix0 = floor_i32(fx)
                        iy0 = floor_i32(fy)
                        dx = fx - ix0.astype(jnp.float32)
                        dy = fy - iy0.astype(jnp.float32)
                        one = jnp.float32(1.0)
                        zero = jnp.float32(0.0)
                        vy0 = iy0 >= 0
                        vy1 = (iy0 + 1) <= H - 1
                        vx0 = ix0 >= 0
                        vx1 = (ix0 + 1) <= W - 1
                        w00 = jnp.where(vy0 & vx0, (one - dy) * (one - dx), zero)
                        w01 = jnp.where(vy0 & vx1, (one - dy) * dx, zero)
                        w10 = jnp.where(vy1 & vx0, dy * (one - dx), zero)
                        w11 = jnp.where(vy1 & vx1, dy * dx, zero)
                        x0c = jnp.clip(ix0, 0, W - 1)
                        x1c = jnp.clip(ix0 + 1, 0, W - 1)
                        xb0 = (x0c // 128) * 1024 + x0c % 128
                        xb1 = (x1c // 128) * 1024 + x1c % 128
                        hi_sel = jnp.where(vy0, hi_off, lo_off)
                        i00 = lo_off + xb0
                        i01 = lo_off + xb1
                        i10 = hi_sel + xb0
                        i11 = hi_sel + xb1
                        for c in range(C):
                            coff = (c // 8) * 4096 + (c % 8) * 128
                            a = (w00 * plsc.load_gather(slab, [i00 + coff])
                                 + w01 * plsc.load_gather(slab, [i01 + coff])
                                 + w10 * plsc.load_gather(slab, [i10 + coff])
                                 + w11 * plsc.load_gather(slab, [i11 + coff]))
                            stag[pl.ds(c * 17, 16)] = a
                        mi = jnp.minimum(jnp.int32(16), m - v * 16)
                        safe = jnp.where(lane < jnp.full((16,), mi, jnp.int32),
                                         jnp.clip(pidf.astype(jnp.int32),
                                                  0, n_pts - 1),
                                         jnp.full((16,), n_pts, jnp.int32) + lane)
                        sidx[0, pl.ds(v * 16, 16)] = safe

                        def transpose_pt(i, c4):
                            for jj in range(C // 16):
                                pr = plsc.load_gather(
                                    stag, [(jj * 16 + lane) * 17 + i])
                                sbuf[v * 16 + i, pl.ds(jj * 16, 16)] = pr
                            return c4

                        lax.fori_loop(0, mi, transpose_pt, 0)
                        return c3

                    lax.fori_loop(0, (m + 15) // 16, do_vec, 0)
                    pltpu.async_copy(sbuf, out_hbm.at[sidx.at[0]], sem).wait()
                    return c2

                lax.fori_loop(0, (n + 63) // 64, do_chunk, 0)

                @pl.when((kk <= band - 2) & (y + 2 <= H - 1))
                def _():
                    pltpu.async_copy(
                        feat_hbm.at[pl.ds(slab0 + (y + 2) * slab_w, slab_w)],
                        slab.at[pl.ds((y % 2) * slab_w, slab_w)], sem2)
                return carry

            lax.fori_loop(0, band, do_bin, 0)

        pl.run_scoped(
            phase_b,
            pltpu.VMEM((2 * slab_w,), jnp.float32),
            pltpu.VMEM((64, 8), jnp.float32),
            pltpu.VMEM((C * 17,), jnp.float32),
            pltpu.VMEM((64, C), jnp.float32),
            pltpu.VMEM((1, 64), jnp.int32),
        )

    return point_sample


def kernel(features, grid):
    B, H, W, C = features.shape
    P = grid.shape[1]
    ftt = (features.reshape(B, H, W // 128, 128, C // 8, 8)
           .transpose(0, 1, 4, 2, 5, 3)
           .reshape(B * H * (C // 8) * (W // 128) * 8 * 128))
    gt = (grid.reshape(B, P // 128, 128, 2)
          .transpose(0, 1, 3, 2)
          .reshape(B * (P // 128), 256))
    out, _ = _build(B, H, W, C, P)(ftt, gt)
    return out[:B * P].reshape(B, P, C)


# speedup vs baseline: 1.5232x; 1.5232x over previous
import functools

import jax
import jax.numpy as jnp
from jax import lax
from jax.experimental import pallas as pl
from jax.experimental.pallas import tpu as pltpu
from jax.experimental.pallas import tpu_sc as plsc

_LANES = 16


def _build(B, H, W, C, P):
    NC, NS = 2, 16
    n_pts = B * P
    sc_pts = n_pts // NC
    ppw = sc_pts // NS
    nbins = (B // NC) * H
    band = nbins // NS
    slab_w = (C // 8) * (W // 128) * 1024
    cg = C // 8
    assert H % 128 == 0 and W % 128 == 0 and C % 8 == 0 and P % 128 == 0

    mesh = plsc.VectorSubcoreMesh(
        core_axis_name="c", subcore_axis_name="s", num_cores=NC, num_subcores=NS)

    @functools.partial(
        pl.kernel,
        out_type=(jax.ShapeDtypeStruct((n_pts + 16, C), jnp.float32),
                  jax.ShapeDtypeStruct((n_pts + 128, 8), jnp.float32)),
        mesh=mesh,
        compiler_params=pltpu.CompilerParams(
            needs_layout_passes=False, use_tc_tiling_on_sc=False),
        scratch_types=[
            pltpu.VMEM((nbins + 16,), jnp.int32),
            pltpu.VMEM((nbins,), jnp.int32),
            pltpu.SMEM((nbins + 16,), jnp.int32),
            pltpu.VMEM_SHARED((NS, nbins), jnp.int32),
            pltpu.SemaphoreType.DMA,
            pltpu.SemaphoreType.DMA,
        ],
    )
    def point_sample(feat_hbm, grid_hbm, out_hbm, rec_hbm,
                     binstart, cursor, binsmem, shared, sem, sem2):
        cid = lax.axis_index("c")
        sid = lax.axis_index("s")
        b_local = sid // (NS // (B // NC))
        pt0_sc = cid * sc_pts
        chunk0 = pt0_sc + sid * ppw
        lane = lax.iota(jnp.int32, 16)
        ones = jnp.ones((16,), jnp.int32)
        fW = jnp.float32(W)
        fH = jnp.float32(H)

        def floor_i32(f):
            t = f.astype(jnp.int32)
            return t - jnp.where(t.astype(jnp.float32) > f, 1, 0)

        def phase_a(gchunk, hist, histall, recs, pos):
            for v in range(nbins // 16):
                hist[pl.ds(v * 16, 16)] = jnp.zeros((16,), jnp.int32)
            pltpu.sync_copy(grid_hbm.at[pl.ds(chunk0 // 128, ppw // 128)], gchunk)

            def keys_of(v):
                row = v // 8
                o = (v % 8) * 16
                gx = gchunk[row, pl.ds(o, 16)]
                gy = gchunk[row, pl.ds(128 + o, 16)]
                iy0 = floor_i32(gy * fH - 0.5)
                k = b_local * H + jnp.clip(iy0, 0, H - 1)
                return gx, gy, k

            def pass1(v, carry):
                _, _, k = keys_of(v)
                plsc.addupdate_scatter(hist, [k], ones)
                return carry

            lax.fori_loop(0, ppw // 16, pass1, 0)
            pltpu.sync_copy(hist, shared.at[sid])
            plsc.subcore_barrier()
            pltpu.sync_copy(shared, histall)

            def prefix(v, run):
                sl = pl.ds(v * 16, 16)
                tot = jnp.zeros((16,), jnp.int32)
                pre = jnp.zeros((16,), jnp.int32)
                for t in range(NS):
                    hv = histall[t, sl]
                    tot = tot + hv
                    pre = pre + jnp.where(jnp.full((16,), t, jnp.int32)
                                          < jnp.full((16,), sid, jnp.int32),
                                          hv, jnp.zeros((16,), jnp.int32))
                excl = plsc.cumsum(tot) - tot
                bsv = excl + run
                binstart[sl] = bsv
                cursor[sl] = bsv + pre
                for i in range(16):
                    binsmem[v * 16 + i] = bsv[i]
                return run + jnp.sum(tot, axis=0)

            lax.fori_loop(0, nbins // 16, prefix, jnp.int32(0))
            binsmem[nbins] = jnp.int32(sc_pts)

            def pass2(v, carry):
                gx, gy, k = keys_of(v)
                rank, _ = plsc.scan_count(k)
                basek = plsc.load_gather(cursor, [k])
                p = basek + rank - 1
                plsc.addupdate_scatter(cursor, [k], ones)
                pos[v // 8, pl.ds((v % 8) * 16, 16)] = (
                    jnp.clip(p, 0, sc_pts - 1) + pt0_sc)
                pid = (chunk0 + v * 16 + lane).astype(jnp.float32)
                pt = v * 16 + lane
                plsc.store_scatter(recs, [pt, jnp.zeros((16,), jnp.int32)], gx)
                plsc.store_scatter(recs, [pt, ones], gy)
                plsc.store_scatter(recs, [pt, ones + ones], pid)
                return carry

            lax.fori_loop(0, ppw // 16, pass2, 0)
            descs = [
                pltpu.async_copy(recs.at[pl.ds(ch * 128, 128)],
                                 rec_hbm.at[pos.at[ch]], sem)
                for ch in range(ppw // 128)
            ]
            for d in descs:
                d.wait()

        pl.run_scoped(
            phase_a,
            pltpu.VMEM((ppw // 128, 256), jnp.float32),
            pltpu.VMEM((nbins,), jnp.int32),
            pltpu.VMEM((NS, nbins), jnp.int32),
            pltpu.VMEM((ppw, 8), jnp.float32),
            pltpu.VMEM((ppw // 128, 128), jnp.int32),
        )
        plsc.subcore_barrier()

        k0 = sid * band
        y0 = (sid % (NS // (B // NC))) * band
        bb = cid * (B // NC) + b_local
        slab0 = (bb * H) * slab_w

        def phase_b(slab, recst, stag, sbuf, sidx):
            pltpu.sync_copy(
                feat_hbm.at[pl.ds(slab0 + y0 * slab_w, slab_w)],
                slab.at[pl.ds((y0 % 2) * slab_w, slab_w)])
            pltpu.async_copy(
                feat_hbm.at[pl.ds(slab0 + (y0 + 1) * slab_w, slab_w)],
                slab.at[pl.ds(((y0 + 1) % 2) * slab_w, slab_w)], sem2)

            def do_bin(kk, carry):
                y = y0 + kk

                @pl.when(y + 1 <= H - 1)
                def _():
                    pltpu.make_async_copy(
                        feat_hbm.at[pl.ds(0, slab_w)],
                        slab.at[pl.ds(((y + 1) % 2) * slab_w, slab_w)],
                        sem2).wait()

                cur = binsmem[k0 + kk]
                nxt = binsmem[k0 + kk + 1]
                n = jnp.clip(nxt - cur, 0, sc_pts)
                gstart = pt0_sc + cur
                lo_off = jnp.full((16,), (y % 2) * slab_w, jnp.int32)
                hi_off = jnp.full((16,), ((y + 1) % 2) * slab_w, jnp.int32)

                def do_chunk(ch, c2):
                    pltpu.sync_copy(rec_hbm.at[pl.ds(gstart + ch * 64, 64)],
                                    recst)
                    m = jnp.minimum(jnp.int32(64), n - ch * 64)

                    def do_vec(v, c3):
                        pt = v * 16 + lane
                        zz = jnp.zeros((16,), jnp.int32)
                        gx = plsc.load_gather(recst, [pt, zz])
                        gy = plsc.load_gather(recst, [pt, zz + 1])
                        pidf = plsc.load_gather(recst, [pt, zz + 2])
                        fx = gx * fW - 0.5
                        fy = gy * fH - 0.5
                        ix0 = floor_i32(fx)
                        iy0 = floor_i32(fy)
                        dx = fx - ix0.astype(jnp.float32)
                        dy = fy - iy0.astype(jnp.float32)
                        one = jnp.float32(1.0)
                        zero = jnp.float32(0.0)
                        vy0 = iy0 >= 0
                        vy1 = (iy0 + 1) <= H - 1
                        vx0 = ix0 >= 0
                        vx1 = (ix0 + 1) <= W - 1
                        w00 = jnp.where(vy0 & vx0, (one - dy) * (one - dx), zero)
                        w01 = jnp.where(vy0 & vx1, (one - dy) * dx, zero)
                        w10 = jnp.where(vy1 & vx0, dy * (one - dx), zero)
                        w11 = jnp.where(vy1 & vx1, dy * dx, zero)
                        x0c = jnp.clip(ix0, 0, W - 1)
                        x1c = jnp.clip(ix0 + 1, 0, W - 1)
                        xb0 = (x0c // 128) * 1024 + x0c % 128
                        xb1 = (x1c // 128) * 1024 + x1c % 128
                        hi_sel = jnp.where(vy0, hi_off, lo_off)
                        i00 = lo_off + xb0
                        i01 = lo_off + xb1
                        i10 = hi_sel + xb0
                        i11 = hi_sel + xb1
                        for c in range(C):
                            coff = (c // 8) * 4096 + (c % 8) * 128
                            a = (w00 * plsc.load_gather(slab, [i00 + coff])
                                 + w01 * plsc.load_gather(slab, [i01 + coff])
                                 + w10 * plsc.load_gather(slab, [i10 + coff])
                                 + w11 * plsc.load_gather(slab, [i11 + coff]))
                            stag[pl.ds(c * 17, 16)] = a
                        mi = jnp.minimum(jnp.int32(16), m - v * 16)
                        safe = jnp.where(lane < jnp.full((16,), mi, jnp.int32),
                                         jnp.clip(pidf.astype(jnp.int32),
                                                  0, n_pts - 1),
                                         jnp.full((16,), n_pts, jnp.int32) + lane)
                        sidx[0] = safe

                        def transpose_pt(i, c4):
                            for jj in range(C // 16):
                                pr = plsc.load_gather(
                                    stag, [(jj * 16 + lane) * 17 + i])
                                sbuf[i, pl.ds(jj * 16, 16)] = pr
                            return c4

                        lax.fori_loop(0, mi, transpose_pt, 0)
                        pltpu.async_copy(sbuf, out_hbm.at[sidx.at[0]], sem).wait()
                        return c3

                    lax.fori_loop(0, (m + 15) // 16, do_vec, 0)
                    return c2

                lax.fori_loop(0, (n + 63) // 64, do_chunk, 0)

                @pl.when((kk <= band - 2) & (y + 2 <= H - 1))
                def _():
                    pltpu.async_copy(
                        feat_hbm.at[pl.ds(slab0 + (y + 2) * slab_w, slab_w)],
                        slab.at[pl.ds((y % 2) * slab_w, slab_w)], sem2)
                return carry

            lax.fori_loop(0, band, do_bin, 0)

        pl.run_scoped(
            phase_b,
            pltpu.VMEM((2 * slab_w,), jnp.float32),
            pltpu.VMEM((64, 8), jnp.float32),
            pltpu.VMEM((C * 17,), jnp.float32),
            pltpu.VMEM((16, C), jnp.float32),
            pltpu.VMEM((1, 16), jnp.int32),
        )

    return point_sample


def kernel(features, grid):
    B, H, W, C = features.shape
    P = grid.shape[1]
    ftt = (features.reshape(B, H, W // 128, 128, C // 8, 8)
           .transpose(0, 1, 4, 2, 5, 3)
           .reshape(B * H * (C // 8) * (W // 128) * 8 * 128))
    gt = (grid.reshape(B, P // 128, 128, 2)
          .transpose(0, 1, 3, 2)
          .reshape(B * (P // 128), 256))
    out, _ = _build(B, H, W, C, P)(ftt, gt)
    return out[:B * P].reshape(B, P, C)


# speedup vs baseline: 1.5838x; 1.0398x over previous
import functools

import jax
import jax.numpy as jnp
from jax import lax
from jax.experimental import pallas as pl
from jax.experimental.pallas import tpu as pltpu
from jax.experimental.pallas import tpu_sc as plsc

_LANES = 16


def _build(B, H, W, C, P):
    NC, NS = 2, 16
    n_pts = B * P
    sc_pts = n_pts // NC
    ppw = sc_pts // NS
    nbins = (B // NC) * H
    band = nbins // NS
    slab_w = (C // 8) * (W // 128) * 1024
    cg = C // 8
    assert H % 128 == 0 and W % 128 == 0 and C % 8 == 0 and P % 128 == 0

    mesh = plsc.VectorSubcoreMesh(
        core_axis_name="c", subcore_axis_name="s", num_cores=NC, num_subcores=NS)

    @functools.partial(
        pl.kernel,
        out_type=(jax.ShapeDtypeStruct((n_pts + NC * NS * 64, C), jnp.float32),
                  jax.ShapeDtypeStruct((n_pts + 128, 8), jnp.float32)),
        mesh=mesh,
        compiler_params=pltpu.CompilerParams(
            needs_layout_passes=False, use_tc_tiling_on_sc=False),
        scratch_types=[
            pltpu.VMEM((nbins + 16,), jnp.int32),
            pltpu.VMEM((nbins,), jnp.int32),
            pltpu.SMEM((nbins + 16,), jnp.int32),
            pltpu.VMEM_SHARED((NS, nbins), jnp.int32),
            pltpu.SemaphoreType.DMA,
            pltpu.SemaphoreType.DMA,
        ],
    )
    def point_sample(feat_hbm, grid_hbm, out_hbm, rec_hbm,
                     binstart, cursor, binsmem, shared, sem, sem2):
        cid = lax.axis_index("c")
        sid = lax.axis_index("s")
        b_local = sid // (NS // (B // NC))
        pt0_sc = cid * sc_pts
        chunk0 = pt0_sc + sid * ppw
        lane = lax.iota(jnp.int32, 16)
        ones = jnp.ones((16,), jnp.int32)
        fW = jnp.float32(W)
        fH = jnp.float32(H)

        def floor_i32(f):
            t = f.astype(jnp.int32)
            return t - jnp.where(t.astype(jnp.float32) > f, 1, 0)

        def phase_a(gchunk, hist, histall, recs, pos):
            for v in range(nbins // 16):
                hist[pl.ds(v * 16, 16)] = jnp.zeros((16,), jnp.int32)
            pltpu.sync_copy(grid_hbm.at[pl.ds(chunk0 // 128, ppw // 128)], gchunk)

            def keys_of(v):
                row = v // 8
                o = (v % 8) * 16
                gx = gchunk[row, pl.ds(o, 16)]
                gy = gchunk[row, pl.ds(128 + o, 16)]
                iy0 = floor_i32(gy * fH - 0.5)
                k = b_local * H + jnp.clip(iy0, 0, H - 1)
                return gx, gy, k

            def pass1(v, carry):
                _, _, k = keys_of(v)
                plsc.addupdate_scatter(hist, [k], ones)
                return carry

            lax.fori_loop(0, ppw // 16, pass1, 0)
            pltpu.sync_copy(hist, shared.at[sid])
            plsc.subcore_barrier()
            pltpu.sync_copy(shared, histall)

            def prefix(v, run):
                sl = pl.ds(v * 16, 16)
                tot = jnp.zeros((16,), jnp.int32)
                pre = jnp.zeros((16,), jnp.int32)
                for t in range(NS):
                    hv = histall[t, sl]
                    tot = tot + hv
                    pre = pre + jnp.where(jnp.full((16,), t, jnp.int32)
                                          < jnp.full((16,), sid, jnp.int32),
                                          hv, jnp.zeros((16,), jnp.int32))
                excl = plsc.cumsum(tot) - tot
                bsv = excl + run
                binstart[sl] = bsv
                cursor[sl] = bsv + pre
                for i in range(16):
                    binsmem[v * 16 + i] = bsv[i]
                return run + jnp.sum(tot, axis=0)

            lax.fori_loop(0, nbins // 16, prefix, jnp.int32(0))
            binsmem[nbins] = jnp.int32(sc_pts)

            def pass2(v, carry):
                gx, gy, k = keys_of(v)
                rank, _ = plsc.scan_count(k)
                basek = plsc.load_gather(cursor, [k])
                p = basek + rank - 1
                plsc.addupdate_scatter(cursor, [k], ones)
                pos[v // 8, pl.ds((v % 8) * 16, 16)] = (
                    jnp.clip(p, 0, sc_pts - 1) + pt0_sc)
                pid = (chunk0 + v * 16 + lane).astype(jnp.float32)
                pt = v * 16 + lane
                plsc.store_scatter(recs, [pt, jnp.zeros((16,), jnp.int32)], gx)
                plsc.store_scatter(recs, [pt, ones], gy)
                plsc.store_scatter(recs, [pt, ones + ones], pid)
                return carry

            lax.fori_loop(0, ppw // 16, pass2, 0)
            descs = [
                pltpu.async_copy(recs.at[pl.ds(ch * 128, 128)],
                                 rec_hbm.at[pos.at[ch]], sem)
                for ch in range(ppw // 128)
            ]
            for d in descs:
                d.wait()

        pl.run_scoped(
            phase_a,
            pltpu.VMEM((ppw // 128, 256), jnp.float32),
            pltpu.VMEM((nbins,), jnp.int32),
            pltpu.VMEM((NS, nbins), jnp.int32),
            pltpu.VMEM((ppw, 8), jnp.float32),
            pltpu.VMEM((ppw // 128, 128), jnp.int32),
        )
        plsc.subcore_barrier()

        k0 = sid * band
        y0 = (sid % (NS // (B // NC))) * band
        bb = cid * (B // NC) + b_local
        slab0 = (bb * H) * slab_w

        def phase_b(slab, recst, stag, sbuf, sidx):
            pltpu.sync_copy(
                feat_hbm.at[pl.ds(slab0 + y0 * slab_w, slab_w)],
                slab.at[pl.ds((y0 % 2) * slab_w, slab_w)])
            pltpu.async_copy(
                feat_hbm.at[pl.ds(slab0 + (y0 + 1) * slab_w, slab_w)],
                slab.at[pl.ds(((y0 + 1) % 2) * slab_w, slab_w)], sem2)

            def do_bin(kk, carry):
                y = y0 + kk

                @pl.when(y + 1 <= H - 1)
                def _():
                    pltpu.make_async_copy(
                        feat_hbm.at[pl.ds(0, slab_w)],
                        slab.at[pl.ds(((y + 1) % 2) * slab_w, slab_w)],
                        sem2).wait()

                cur = binsmem[k0 + kk]
                nxt = binsmem[k0 + kk + 1]
                n = jnp.clip(nxt - cur, 0, sc_pts)
                gstart = pt0_sc + cur
                lo_off = jnp.full((16,), (y % 2) * slab_w, jnp.int32)
                hi_off = jnp.full((16,), ((y + 1) % 2) * slab_w, jnp.int32)

                def do_chunk(ch, c2):
                    pltpu.sync_copy(rec_hbm.at[pl.ds(gstart + ch * 64, 64)],
                                    recst)
                    m = jnp.minimum(jnp.int32(64), n - ch * 64)
                    dumb = n_pts + (sid * NC + cid) * 64
                    for vv in range(4):
                        sidx[0, pl.ds(vv * 16, 16)] = (
                            jnp.full((16,), dumb + vv * 16, jnp.int32) + lane)

                    def do_vec(v, c3):
                        pt = v * 16 + lane
                        zz = jnp.zeros((16,), jnp.int32)
                        gx = plsc.load_gather(recst, [pt, zz])
                        gy = plsc.load_gather(recst, [pt, zz + 1])
                        pidf = plsc.load_gather(recst, [pt, zz + 2])
                        fx = gx * fW - 0.5
                        fy = gy * fH - 0.5
                        ix0 = floor_i32(fx)
                        iy0 = floor_i32(fy)
                        dx = fx - ix0.astype(jnp.float32)
                        dy = fy - iy0.astype(jnp.float32)
                        one = jnp.float32(1.0)
                        zero = jnp.float32(0.0)
                        vy0 = iy0 >= 0
                        vy1 = (iy0 + 1) <= H - 1
                        vx0 = ix0 >= 0
                        vx1 = (ix0 + 1) <= W - 1
                        w00 = jnp.where(vy0 & vx0, (one - dy) * (one - dx), zero)
                        w01 = jnp.where(vy0 & vx1, (one - dy) * dx, zero)
                        w10 = jnp.where(vy1 & vx0, dy * (one - dx), zero)
                        w11 = jnp.where(vy1 & vx1, dy * dx, zero)
                        x0c = jnp.clip(ix0, 0, W - 1)
                        x1c = jnp.clip(ix0 + 1, 0, W - 1)
                        xb0 = (x0c // 128) * 1024 + x0c % 128
                        xb1 = (x1c // 128) * 1024 + x1c % 128
                        hi_sel = jnp.where(vy0, hi_off, lo_off)
                        i00 = lo_off + xb0
                        i01 = lo_off + xb1
                        i10 = hi_sel + xb0
                        i11 = hi_sel + xb1
                        for c in range(C):
                            coff = (c // 8) * 4096 + (c % 8) * 128
                            a = (w00 * plsc.load_gather(slab, [i00 + coff])
                                 + w01 * plsc.load_gather(slab, [i01 + coff])
                                 + w10 * plsc.load_gather(slab, [i10 + coff])
                                 + w11 * plsc.load_gather(slab, [i11 + coff]))
                            stag[pl.ds(c * 17, 16)] = a
                        mi = jnp.minimum(jnp.int32(16), m - v * 16)
                        safe = jnp.where(
                            lane < jnp.full((16,), mi, jnp.int32),
                            jnp.clip(pidf.astype(jnp.int32), 0, n_pts - 1),
                            jnp.full((16,), dumb + v * 16, jnp.int32) + lane)
                        sidx[0, pl.ds(v * 16, 16)] = safe

                        def transpose_pt(i, c4):
                            for jj in range(C // 16):
                                pr = plsc.load_gather(
                                    stag, [(jj * 16 + lane) * 17 + i])
                                sbuf[v * 16 + i, pl.ds(jj * 16, 16)] = pr
                            return c4

                        lax.fori_loop(0, mi, transpose_pt, 0)
                        return c3

                    lax.fori_loop(0, (m + 15) // 16, do_vec, 0)
                    pltpu.async_copy(sbuf, out_hbm.at[sidx.at[0]], sem).wait()
                    return c2

                lax.fori_loop(0, (n + 63) // 64, do_chunk, 0)

                @pl.when((kk <= band - 2) & (y + 2 <= H - 1))
                def _():
                    pltpu.async_copy(
                        feat_hbm.at[pl.ds(slab0 + (y + 2) * slab_w, slab_w)],
                        slab.at[pl.ds((y % 2) * slab_w, slab_w)], sem2)
                return carry

            lax.fori_loop(0, band, do_bin, 0)

        pl.run_scoped(
            phase_b,
            pltpu.VMEM((2 * slab_w,), jnp.float32),
            pltpu.VMEM((64, 8), jnp.float32),
            pltpu.VMEM((C * 17,), jnp.float32),
            pltpu.VMEM((64, C), jnp.float32),
            pltpu.VMEM((1, 64), jnp.int32),
        )

    return point_sample


def kernel(features, grid):
    B, H, W, C = features.shape
    P = grid.shape[1]
    ftt = (features.reshape(B, H, W // 128, 128, C // 8, 8)
           .transpose(0, 1, 4, 2, 5, 3)
           .reshape(B * H * (C // 8) * (W // 128) * 8 * 128))
    gt = (grid.reshape(B, P // 128, 128, 2)
          .transpose(0, 1, 3, 2)
          .reshape(B * (P // 128), 256))
    out, _ = _build(B, H, W, C, P)(ftt, gt)
    return out[:B * P].reshape(B, P, C)


# speedup vs baseline: 1.7267x; 1.0903x over previous
import functools

import jax
import jax.numpy as jnp
from jax import lax
from jax.experimental import pallas as pl
from jax.experimental.pallas import tpu as pltpu
from jax.experimental.pallas import tpu_sc as plsc

_LANES = 16


def _build(B, H, W, C, P):
    NC, NS = 2, 16
    n_pts = B * P
    sc_pts = n_pts // NC
    ppw = sc_pts // NS
    nbins = (B // NC) * H
    band = nbins // NS
    slab_w = (C // 8) * (W // 128) * 1024
    cg = C // 8
    assert H % 128 == 0 and W % 128 == 0 and C % 8 == 0 and P % 128 == 0

    mesh = plsc.VectorSubcoreMesh(
        core_axis_name="c", subcore_axis_name="s", num_cores=NC, num_subcores=NS)

    @functools.partial(
        pl.kernel,
        out_type=(jax.ShapeDtypeStruct((n_pts + NC * NS * 64, C), jnp.float32),
                  jax.ShapeDtypeStruct((n_pts + 128, 8), jnp.float32)),
        mesh=mesh,
        compiler_params=pltpu.CompilerParams(
            needs_layout_passes=False, use_tc_tiling_on_sc=False),
        scratch_types=[
            pltpu.VMEM((nbins + 16,), jnp.int32),
            pltpu.VMEM((nbins,), jnp.int32),
            pltpu.SMEM((nbins + 16,), jnp.int32),
            pltpu.VMEM_SHARED((NS, nbins), jnp.int32),
            pltpu.SemaphoreType.DMA,
            pltpu.SemaphoreType.DMA,
            pltpu.SemaphoreType.DMA,
        ],
    )
    def point_sample(feat_hbm, grid_hbm, out_hbm, rec_hbm,
                     binstart, cursor, binsmem, shared, sem, sem2, sem3):
        cid = lax.axis_index("c")
        sid = lax.axis_index("s")
        b_local = sid // (NS // (B // NC))
        pt0_sc = cid * sc_pts
        chunk0 = pt0_sc + sid * ppw
        lane = lax.iota(jnp.int32, 16)
        ones = jnp.ones((16,), jnp.int32)
        fW = jnp.float32(W)
        fH = jnp.float32(H)

        def floor_i32(f):
            t = f.astype(jnp.int32)
            return t - jnp.where(t.astype(jnp.float32) > f, 1, 0)

        def phase_a(gchunk, hist, histall, recs, pos):
            for v in range(nbins // 16):
                hist[pl.ds(v * 16, 16)] = jnp.zeros((16,), jnp.int32)
            pltpu.sync_copy(grid_hbm.at[pl.ds(chunk0 // 128, ppw // 128)], gchunk)

            def keys_of(v):
                row = v // 8
                o = (v % 8) * 16
                gx = gchunk[row, pl.ds(o, 16)]
                gy = gchunk[row, pl.ds(128 + o, 16)]
                iy0 = floor_i32(gy * fH - 0.5)
                k = b_local * H + jnp.clip(iy0, 0, H - 1)
                return gx, gy, k

            def pass1(v, carry):
                _, _, k = keys_of(v)
                plsc.addupdate_scatter(hist, [k], ones)
                return carry

            lax.fori_loop(0, ppw // 16, pass1, 0)
            pltpu.sync_copy(hist, shared.at[sid])
            plsc.subcore_barrier()
            pltpu.sync_copy(shared, histall)

            def prefix(v, run):
                sl = pl.ds(v * 16, 16)
                tot = jnp.zeros((16,), jnp.int32)
                pre = jnp.zeros((16,), jnp.int32)
                for t in range(NS):
                    hv = histall[t, sl]
                    tot = tot + hv
                    pre = pre + jnp.where(jnp.full((16,), t, jnp.int32)
                                          < jnp.full((16,), sid, jnp.int32),
                                          hv, jnp.zeros((16,), jnp.int32))
                excl = plsc.cumsum(tot) - tot
                bsv = excl + run
                binstart[sl] = bsv
                cursor[sl] = bsv + pre
                for i in range(16):
                    binsmem[v * 16 + i] = bsv[i]
                return run + jnp.sum(tot, axis=0)

            lax.fori_loop(0, nbins // 16, prefix, jnp.int32(0))
            binsmem[nbins] = jnp.int32(sc_pts)

            def pass2(v, carry):
                gx, gy, k = keys_of(v)
                rank, _ = plsc.scan_count(k)
                basek = plsc.load_gather(cursor, [k])
                p = basek + rank - 1
                plsc.addupdate_scatter(cursor, [k], ones)
                pos[v // 8, pl.ds((v % 8) * 16, 16)] = (
                    jnp.clip(p, 0, sc_pts - 1) + pt0_sc)
                pid = (chunk0 + v * 16 + lane).astype(jnp.float32)
                pt = v * 16 + lane
                plsc.store_scatter(recs, [pt, jnp.zeros((16,), jnp.int32)], gx)
                plsc.store_scatter(recs, [pt, ones], gy)
                plsc.store_scatter(recs, [pt, ones + ones], pid)
                return carry

            lax.fori_loop(0, ppw // 16, pass2, 0)
            descs = [
                pltpu.async_copy(recs.at[pl.ds(ch * 128, 128)],
                                 rec_hbm.at[pos.at[ch]], sem)
                for ch in range(ppw // 128)
            ]
            for d in descs:
                d.wait()

        pl.run_scoped(
            phase_a,
            pltpu.VMEM((ppw // 128, 256), jnp.float32),
            pltpu.VMEM((nbins,), jnp.int32),
            pltpu.VMEM((NS, nbins), jnp.int32),
            pltpu.VMEM((ppw, 8), jnp.float32),
            pltpu.VMEM((ppw // 128, 128), jnp.int32),
        )
        plsc.subcore_barrier()

        k0 = sid * band
        y0 = (sid % (NS // (B // NC))) * band
        bb = cid * (B // NC) + b_local
        slab0 = (bb * H) * slab_w

        def phase_b(slab, recst, stag, sbuf, sidx):
            pltpu.sync_copy(
                feat_hbm.at[pl.ds(slab0 + y0 * slab_w, slab_w)],
                slab.at[pl.ds((y0 % 2) * slab_w, slab_w)])
            pltpu.async_copy(
                feat_hbm.at[pl.ds(slab0 + (y0 + 1) * slab_w, slab_w)],
                slab.at[pl.ds(((y0 + 1) % 2) * slab_w, slab_w)], sem2)
            pltpu.sync_copy(
                rec_hbm.at[pl.ds(pt0_sc + binsmem[k0], 64)], recst.at[0])

            def do_bin(kk, carry):
                y = y0 + kk
                par = kk % 2

                @pl.when(kk > 0)
                def _():
                    pltpu.make_async_copy(
                        rec_hbm.at[pl.ds(0, 64)], recst.at[par], sem3).wait()

                @pl.when(y + 1 <= H - 1)
                def _():
                    pltpu.make_async_copy(
                        feat_hbm.at[pl.ds(0, slab_w)],
                        slab.at[pl.ds(((y + 1) % 2) * slab_w, slab_w)],
                        sem2).wait()

                cur = binsmem[k0 + kk]
                nxt = binsmem[k0 + kk + 1]
                n = jnp.clip(nxt - cur, 0, sc_pts)
                gstart = pt0_sc + cur
                lo_off = jnp.full((16,), (y % 2) * slab_w, jnp.int32)
                hi_off = jnp.full((16,), ((y + 1) % 2) * slab_w, jnp.int32)

                def do_chunk(ch, c2):
                    @pl.when(ch > 0)
                    def _():
                        pltpu.sync_copy(
                            rec_hbm.at[pl.ds(gstart + ch * 64, 64)],
                            recst.at[par])

                    m = jnp.minimum(jnp.int32(64), n - ch * 64)
                    dumb = n_pts + (sid * NC + cid) * 64
                    for vv in range(4):
                        sidx[0, pl.ds(vv * 16, 16)] = (
                            jnp.full((16,), dumb + vv * 16, jnp.int32) + lane)

                    def do_vec(v, c3):
                        pt = v * 16 + lane
                        zz = jnp.zeros((16,), jnp.int32)
                        pv = jnp.full((16,), par, jnp.int32)
                        gx = plsc.load_gather(recst, [pv, pt, zz])
                        gy = plsc.load_gather(recst, [pv, pt, zz + 1])
                        pidf = plsc.load_gather(recst, [pv, pt, zz + 2])
                        fx = gx * fW - 0.5
                        fy = gy * fH - 0.5
                        ix0 = floor_i32(fx)
                        iy0 = floor_i32(fy)
                        dx = fx - ix0.astype(jnp.float32)
                        dy = fy - iy0.astype(jnp.float32)
                        one = jnp.float32(1.0)
                        zero = jnp.float32(0.0)
                        vy0 = iy0 >= 0
                        vy1 = (iy0 + 1) <= H - 1
                        vx0 = ix0 >= 0
                        vx1 = (ix0 + 1) <= W - 1
                        w00 = jnp.where(vy0 & vx0, (one - dy) * (one - dx), zero)
                        w01 = jnp.where(vy0 & vx1, (one - dy) * dx, zero)
                        w10 = jnp.where(vy1 & vx0, dy * (one - dx), zero)
                        w11 = jnp.where(vy1 & vx1, dy * dx, zero)
                        x0c = jnp.clip(ix0, 0, W - 1)
                        x1c = jnp.clip(ix0 + 1, 0, W - 1)
                        xb0 = (x0c // 128) * 1024 + x0c % 128
                        xb1 = (x1c // 128) * 1024 + x1c % 128
                        hi_sel = jnp.where(vy0, hi_off, lo_off)
                        i00 = lo_off + xb0
                        i01 = lo_off + xb1
                        i10 = hi_sel + xb0
                        i11 = hi_sel + xb1
                        for c in range(C):
                            coff = (c // 8) * 4096 + (c % 8) * 128
                            a = (w00 * plsc.load_gather(slab, [i00 + coff])
                                 + w01 * plsc.load_gather(slab, [i01 + coff])
                                 + w10 * plsc.load_gather(slab, [i10 + coff])
                                 + w11 * plsc.load_gather(slab, [i11 + coff]))
                            stag[pl.ds(c * 17, 16)] = a
                        mi = jnp.minimum(jnp.int32(16), m - v * 16)
                        safe = jnp.where(
                            lane < jnp.full((16,), mi, jnp.int32),
                            jnp.clip(pidf.astype(jnp.int32), 0, n_pts - 1),
                            jnp.full((16,), dumb + v * 16, jnp.int32) + lane)
                        sidx[0, pl.ds(v * 16, 16)] = safe

                        def transpose_pt(i, c4):
                            for jj in range(C // 16):
                                pr = plsc.load_gather(
                                    stag, [(jj * 16 + lane) * 17 + i])
                                sbuf[v * 16 + i, pl.ds(jj * 16, 16)] = pr
                            return c4

                        lax.fori_loop(0, mi, transpose_pt, 0)
                        return c3

                    lax.fori_loop(0, (m + 15) // 16, do_vec, 0)
                    pltpu.async_copy(sbuf, out_hbm.at[sidx.at[0]], sem).wait()
                    return c2

                lax.fori_loop(0, (n + 63) // 64, do_chunk, 0)

                @pl.when(kk <= band - 2)
                def _():
                    pltpu.async_copy(
                        rec_hbm.at[pl.ds(pt0_sc + binsmem[k0 + kk + 1], 64)],
                        recst.at[(kk + 1) % 2], sem3)

                @pl.when((kk <= band - 2) & (y + 2 <= H - 1))
                def _():
                    pltpu.async_copy(
                        feat_hbm.at[pl.ds(slab0 + (y + 2) * slab_w, slab_w)],
                        slab.at[pl.ds((y % 2) * slab_w, slab_w)], sem2)
                return carry

            lax.fori_loop(0, band, do_bin, 0)

        pl.run_scoped(
            phase_b,
            pltpu.VMEM((2 * slab_w,), jnp.float32),
            pltpu.VMEM((2, 64, 8), jnp.float32),
            pltpu.VMEM((C * 17,), jnp.float32),
            pltpu.VMEM((64, C), jnp.float32),
            pltpu.VMEM((1, 64), jnp.int32),
        )

    return point_sample


def kernel(features, grid):
    B, H, W, C = features.shape
    P = grid.shape[1]
    ftt = (features.reshape(B, H, W // 128, 128, C // 8, 8)
           .transpose(0, 1, 4, 2, 5, 3)
           .reshape(B * H * (C // 8) * (W // 128) * 8 * 128))
    gt = (grid.reshape(B, P // 128, 128, 2)
          .transpose(0, 1, 3, 2)
          .reshape(B * (P // 128), 256))
    out, _ = _build(B, H, W, C, P)(ftt, gt)
    return out[:B * P].reshape(B, P, C)
